# Initial kernel scaffold; baseline (speedup 1.0000x reference)
import jax
import jax.numpy as jnp
from jax.experimental import pallas as pl


def kernel(x, embedding_matrix):
    raise NotImplementedError("write your pallas kernel here")


# speedup vs baseline: 6.3314x; 6.3314x over previous
import functools

import jax
import jax.numpy as jnp
from jax import lax
from jax.experimental import pallas as pl
from jax.experimental.pallas import tpu as pltpu
from jax.experimental.pallas import tpu_sc as plsc

VOCAB = 100000
EMB_DIM = 128
BATCH = 4096
HIST = 200

NUM_WORKERS = 32
TOTAL = BATCH * HIST
ROWS_PER_W = TOTAL // NUM_WORKERS
CHUNK = 128
NCHUNKS = ROWS_PER_W // CHUNK


def _make_lookup():
  mesh = plsc.VectorSubcoreMesh(core_axis_name="c", subcore_axis_name="s")

  @functools.partial(
      pl.kernel,
      mesh=mesh,
      out_type=jax.ShapeDtypeStruct((TOTAL, EMB_DIM), jnp.float32),
      scratch_types=[
          pltpu.VMEM((ROWS_PER_W,), jnp.int32),
          pltpu.VMEM((CHUNK, EMB_DIM), jnp.float32),
          pltpu.SemaphoreType.DMA,
      ],
  )
  def lookup(table_hbm, idx_hbm, out_hbm, idx_v, rows_v, sem):
    wid = lax.axis_index("s") * 2 + lax.axis_index("c")
    base = wid * ROWS_PER_W
    pltpu.sync_copy(idx_hbm.at[pl.ds(base, ROWS_PER_W)], idx_v)

    def body(j, carry):
      off = j * CHUNK
      pltpu.async_copy(
          table_hbm.at[idx_v.at[pl.ds(off, CHUNK)]], rows_v, sem
      ).wait()
      pltpu.sync_copy(rows_v, out_hbm.at[pl.ds(base + off, CHUNK)])
      return carry

    lax.fori_loop(0, NCHUNKS, body, 0)

  return lookup


_lookup = _make_lookup()


def kernel(x, embedding_matrix):
  idx = x.reshape(TOTAL).astype(jnp.int32)
  out = _lookup(embedding_matrix, idx)
  return out.reshape(BATCH, HIST, EMB_DIM)


# speedup vs baseline: 9.1433x; 1.4441x over previous
import functools

import jax
import jax.numpy as jnp
from jax import lax
from jax.experimental import pallas as pl
from jax.experimental.pallas import tpu as pltpu
from jax.experimental.pallas import tpu_sc as plsc

VOCAB = 100000
EMB_DIM = 128
BATCH = 4096
HIST = 200

NUM_WORKERS = 32
TOTAL = BATCH * HIST
ROWS_PER_W = TOTAL // NUM_WORKERS
CHUNK = 128
NCHUNKS = ROWS_PER_W // CHUNK
NSIDES = 4


def _make_lookup():
  mesh = plsc.VectorSubcoreMesh(core_axis_name="c", subcore_axis_name="s")

  @functools.partial(
      pl.kernel,
      mesh=mesh,
      out_type=jax.ShapeDtypeStruct((TOTAL, EMB_DIM), jnp.float32),
      scratch_types=[
          pltpu.VMEM((ROWS_PER_W,), jnp.int32),
          pltpu.VMEM((CHUNK, EMB_DIM), jnp.float32),
          pltpu.VMEM((CHUNK, EMB_DIM), jnp.float32),
          pltpu.VMEM((CHUNK, EMB_DIM), jnp.float32),
          pltpu.VMEM((CHUNK, EMB_DIM), jnp.float32),
          pltpu.SemaphoreType.DMA,
          pltpu.SemaphoreType.DMA,
          pltpu.SemaphoreType.DMA,
          pltpu.SemaphoreType.DMA,
          pltpu.SemaphoreType.DMA,
          pltpu.SemaphoreType.DMA,
          pltpu.SemaphoreType.DMA,
          pltpu.SemaphoreType.DMA,
      ],
  )
  def lookup(table_hbm, idx_hbm, out_hbm, idx_v, r0, r1, r2, r3,
             g0, g1, g2, g3, s0, s1, s2, s3):
    bufs = (r0, r1, r2, r3)
    gsems = (g0, g1, g2, g3)
    ssems = (s0, s1, s2, s3)

    wid = lax.axis_index("s") * 2 + lax.axis_index("c")
    base = wid * ROWS_PER_W
    pltpu.sync_copy(idx_hbm.at[pl.ds(base, ROWS_PER_W)], idx_v)

    def g_copy(side, t):
      return pltpu.make_async_copy(
          table_hbm.at[idx_v.at[pl.ds(t * CHUNK, CHUNK)]],
          bufs[side], gsems[side])

    def s_copy(side, t):
      return pltpu.make_async_copy(
          bufs[side], out_hbm.at[pl.ds(base + t * CHUNK, CHUNK)],
          ssems[side])

    def turn(t, side, issue_next):
      prev = (side - 1) % NSIDES
      g_copy(side, t).wait()
      s_copy(side, t).start()
      s_copy(prev, t - 1).wait()
      if issue_next:
        g_copy(prev, t + NSIDES - 1).start()

    for g in range(NSIDES):
      g_copy(g, g).start()

    g_copy(0, 0).wait()
    s_copy(0, 0).start()
    for t in range(1, NSIDES):
      turn(t, t, True)

    def body(k, carry):
      t0 = k * NSIDES
      for b in range(NSIDES):
        turn(t0 + b, b, True)
      return carry

    lax.fori_loop(1, NCHUNKS // NSIDES - 1, body, 0)

    last = NCHUNKS - NSIDES
    turn(last, 0, True)
    for b in range(1, NSIDES):
      turn(last + b, b, False)
    s_copy(NSIDES - 1, NCHUNKS - 1).wait()

  return lookup


_lookup = _make_lookup()


def kernel(x, embedding_matrix):
  idx = x.reshape(TOTAL).astype(jnp.int32)
  out = _lookup(embedding_matrix, idx)
  return out.reshape(BATCH, HIST, EMB_DIM)


# speedup vs baseline: 9.1816x; 1.0042x over previous
import functools

import jax
import jax.numpy as jnp
from jax import lax
from jax.experimental import pallas as pl
from jax.experimental.pallas import tpu as pltpu
from jax.experimental.pallas import tpu_sc as plsc

VOCAB = 100000
EMB_DIM = 128
BATCH = 4096
HIST = 200

NUM_WORKERS = 32
TOTAL = BATCH * HIST
ROWS_PER_W = TOTAL // NUM_WORKERS
CHUNK = 128
NCHUNKS = ROWS_PER_W // CHUNK
NSIDES = 5


def _make_lookup():
  mesh = plsc.VectorSubcoreMesh(core_axis_name="c", subcore_axis_name="s")

  scratch = (
      [pltpu.VMEM((ROWS_PER_W,), jnp.int32)]
      + [pltpu.VMEM((CHUNK, EMB_DIM), jnp.float32)] * NSIDES
      + [pltpu.SemaphoreType.DMA] * (2 * NSIDES)
  )

  @functools.partial(
      pl.kernel,
      mesh=mesh,
      out_type=jax.ShapeDtypeStruct((TOTAL, EMB_DIM), jnp.float32),
      scratch_types=scratch,
  )
  def lookup(table_hbm, idx_hbm, out_hbm, idx_v, *rest):
    bufs = rest[:NSIDES]
    gsems = rest[NSIDES:2 * NSIDES]
    ssems = rest[2 * NSIDES:]

    wid = lax.axis_index("s") * 2 + lax.axis_index("c")
    base = wid * ROWS_PER_W
    pltpu.sync_copy(idx_hbm.at[pl.ds(base, ROWS_PER_W)], idx_v)

    def g_copy(side, t):
      return pltpu.make_async_copy(
          table_hbm.at[idx_v.at[pl.ds(t * CHUNK, CHUNK)]],
          bufs[side], gsems[side])

    def s_copy(side, t):
      return pltpu.make_async_copy(
          bufs[side], out_hbm.at[pl.ds(base + t * CHUNK, CHUNK)],
          ssems[side])

    def turn(t, side, issue_next):
      prev = (side - 1) % NSIDES
      g_copy(side, t).wait()
      s_copy(side, t).start()
      s_copy(prev, t - 1).wait()
      if issue_next:
        g_copy(prev, t + NSIDES - 1).start()

    for g in range(NSIDES):
      g_copy(g, g).start()

    g_copy(0, 0).wait()
    s_copy(0, 0).start()
    for t in range(1, NSIDES):
      turn(t, t, True)

    def body(k, carry):
      t0 = k * NSIDES
      for b in range(NSIDES):
        turn(t0 + b, b, True)
      return carry

    lax.fori_loop(1, NCHUNKS // NSIDES - 1, body, 0)

    last = NCHUNKS - NSIDES
    turn(last, 0, True)
    for b in range(1, NSIDES):
      turn(last + b, b, False)
    s_copy(NSIDES - 1, NCHUNKS - 1).wait()

  return lookup


_lookup = _make_lookup()


def kernel(x, embedding_matrix):
  idx = x.reshape(TOTAL).astype(jnp.int32)
  out = _lookup(embedding_matrix, idx)
  return out.reshape(BATCH, HIST, EMB_DIM)
